# baseline (speedup 1.0000x reference)
import functools
import jax
import jax.numpy as jnp
from jax import lax
from jax.experimental import pallas as pl
from jax.experimental.pallas import tpu as pltpu
from jax.experimental.pallas import tpu_sc as plsc

DEPTH = 32
B = 4
H = 512
W = 512
ROWS = B * H
NW = 32
RPW = ROWS // NW
NLANES = 16


def _sc_body(x_hbm, out_hbm, xrow, obuf):
    wid = lax.axis_index("s") * 2 + lax.axis_index("c")
    lanes = lax.iota(jnp.int32, NLANES)
    ones_v = jnp.full((NLANES,), 1.0, jnp.float32)
    zeros_v = jnp.zeros((NLANES,), jnp.float32)

    def zero_row(j, _):
        def zero_chunk(c, _):
            obuf[j, pl.ds(c * NLANES, NLANES)] = zeros_v
            return 0
        return lax.fori_loop(0, W // NLANES, zero_chunk, 0)
    lax.fori_loop(0, DEPTH, zero_row, 0)

    def row_body(i, _):
        r = wid * RPW + i
        b = r // H
        h = r % H
        pltpu.sync_copy(x_hbm.at[r], xrow)

        def scatter_ones(c, _):
            xv = xrow[pl.ds(c * NLANES, NLANES)]
            col = c * NLANES + lanes
            plsc.store_scatter(obuf, [xv, col], ones_v)
            return 0
        lax.fori_loop(0, W // NLANES, scatter_ones, 0)

        pltpu.sync_copy(obuf, out_hbm.at[b, :, h, :])

        def scatter_zeros(c, _):
            xv = xrow[pl.ds(c * NLANES, NLANES)]
            col = c * NLANES + lanes
            plsc.store_scatter(obuf, [xv, col], zeros_v)
            return 0
        return lax.fori_loop(0, W // NLANES, scatter_zeros, 0)
    lax.fori_loop(0, RPW, row_body, 0)


def kernel(rank, X_in, ones):
    x = X_in.reshape(ROWS, W)
    run = functools.partial(
        pl.kernel,
        out_type=jax.ShapeDtypeStruct((B, DEPTH, H, W), jnp.float32),
        mesh=plsc.VectorSubcoreMesh(core_axis_name="c", subcore_axis_name="s"),
        scratch_types=[
            pltpu.VMEM((W,), jnp.int32),
            pltpu.VMEM((DEPTH, W), jnp.float32),
        ],
        compiler_params=pltpu.CompilerParams(
            use_tc_tiling_on_sc=False, needs_layout_passes=False
        ),
    )(_sc_body)
    return run(x)


# speedup vs baseline: 1.2890x; 1.2890x over previous
import functools
import jax
import jax.numpy as jnp
from jax import lax
from jax.experimental import pallas as pl
from jax.experimental.pallas import tpu as pltpu
from jax.experimental.pallas import tpu_sc as plsc

DEPTH = 32
B = 4
H = 512
W = 512
ROWS = B * H
NW = 32
RPW = ROWS // NW
RB = 2
SR = RPW // RB
NLANES = 16
NCHUNK = W // NLANES


def _sc_body(x_hbm, out_hbm,
             xr0, xr1, xr2, xr3, ob0, ob1,
             xs0, xs1, xs2, xs3, os0, os1):
    xrs = (xr0, xr1, xr2, xr3)
    xss = (xs0, xs1, xs2, xs3)
    obs = (ob0, ob1)
    oss = (os0, os1)
    wid = lax.axis_index("s") * 2 + lax.axis_index("c")
    lanes = lax.iota(jnp.int32, NLANES)
    ones_v = jnp.full((NLANES,), 1.0, jnp.float32)
    zeros_v = jnp.zeros((NLANES,), jnp.float32)

    def row0_of(s):
        return wid * RPW + RB * s

    def start_x(s, slot):
        r0 = row0_of(s)
        pltpu.async_copy(x_hbm.at[pl.ds(r0, RB), :], xrs[slot], xss[slot])

    def wait_x(s, slot):
        r0 = row0_of(s)
        pltpu.make_async_copy(
            x_hbm.at[pl.ds(r0, RB), :], xrs[slot], xss[slot]).wait()

    def out_slice(s):
        r0 = row0_of(s)
        return out_hbm.at[r0 // H, :, pl.ds(r0 % H, RB), :]

    def start_out(s, k):
        pltpu.async_copy(obs[k], out_slice(s), oss[k])

    def wait_out(s, k):
        pltpu.make_async_copy(obs[k], out_slice(s), oss[k]).wait()

    def scatter(k, slot, val):
        ob = obs[k]
        xr = xrs[slot]
        rows = [jnp.full((NLANES,), tr, jnp.int32) for tr in range(RB)]

        def body(c, _):
            col = c * NLANES + lanes
            for tr in range(RB):
                xv = xr[tr, pl.ds(c * NLANES, NLANES)]
                plsc.store_scatter(ob, [xv, rows[tr], col], val)
            return 0
        lax.fori_loop(0, NCHUNK, body, 0, unroll=4)

    def zero_init(ob):
        def per_d(d, _):
            def per_c(c, _):
                for tr in range(RB):
                    ob[d, tr, pl.ds(c * NLANES, NLANES)] = zeros_v
                return 0
            return lax.fori_loop(0, NCHUNK, per_c, 0, unroll=4)
        lax.fori_loop(0, DEPTH, per_d, 0)

    zero_init(ob0)
    zero_init(ob1)
    start_x(0, 0)
    start_x(1, 1)
    start_x(2, 2)
    wait_x(0, 0)
    scatter(0, 0, ones_v)
    start_out(0, 0)
    start_x(3, 3)
    wait_x(1, 1)
    scatter(1, 1, ones_v)
    start_out(1, 1)

    def steady(j, _):
        for t in range(4):
            s = 4 * j + 2 + t
            k = t % 2
            x4 = (2 + t) % 4
            xz = t
            wait_out(s - 2, k)
            scatter(k, xz, zeros_v)
            start_x(s + 2, xz)
            wait_x(s, x4)
            scatter(k, x4, ones_v)
            start_out(s, k)
        return 0
    lax.fori_loop(0, (SR - 4) // 4, steady, 0)

    for s, k, x4, xz in ((SR - 2, 0, 2, 0), (SR - 1, 1, 3, 1)):
        wait_out(s - 2, k)
        scatter(k, xz, zeros_v)
        wait_x(s, x4)
        scatter(k, x4, ones_v)
        start_out(s, k)
    wait_out(SR - 2, 0)
    wait_out(SR - 1, 1)


def kernel(rank, X_in, ones):
    x = X_in.reshape(ROWS, W)
    run = functools.partial(
        pl.kernel,
        out_type=jax.ShapeDtypeStruct((B, DEPTH, H, W), jnp.float32),
        mesh=plsc.VectorSubcoreMesh(core_axis_name="c", subcore_axis_name="s"),
        scratch_types=[
            pltpu.VMEM((RB, W), jnp.int32),
            pltpu.VMEM((RB, W), jnp.int32),
            pltpu.VMEM((RB, W), jnp.int32),
            pltpu.VMEM((RB, W), jnp.int32),
            pltpu.VMEM((DEPTH, RB, W), jnp.float32),
            pltpu.VMEM((DEPTH, RB, W), jnp.float32),
            pltpu.SemaphoreType.DMA,
            pltpu.SemaphoreType.DMA,
            pltpu.SemaphoreType.DMA,
            pltpu.SemaphoreType.DMA,
            pltpu.SemaphoreType.DMA,
            pltpu.SemaphoreType.DMA,
        ],
        compiler_params=pltpu.CompilerParams(
            use_tc_tiling_on_sc=False, needs_layout_passes=False
        ),
    )(_sc_body)
    return run(x)
